# baseline (speedup 1.0000x reference)
import functools

import jax
import jax.numpy as jnp
from jax import lax
from jax.experimental import pallas as pl
from jax.experimental.pallas import tpu as pltpu
from jax.experimental.pallas import tpu_sc as plsc

NC = 2
NS = 16
NW = NC * NS
CHUNK = 128
DPAD = 384


@functools.lru_cache(maxsize=None)
def _make_lookup(n_tokens: int):
    assert n_tokens % (NW * CHUNK) == 0
    b_per_w = n_tokens // NW
    n_chunks = b_per_w // CHUNK
    mesh = plsc.VectorSubcoreMesh(core_axis_name="c", subcore_axis_name="s")

    @functools.partial(
        pl.kernel,
        mesh=mesh,
        out_type=jax.ShapeDtypeStruct((n_tokens, DPAD), jnp.float32),
        scratch_types=[
            pltpu.VMEM((n_chunks, CHUNK), jnp.int32),
            pltpu.VMEM((CHUNK, DPAD), jnp.float32),
            pltpu.SemaphoreType.DMA,
        ],
    )
    def lookup(idx_hbm, table_hbm, out_hbm, idx_v, rows_v, g_sem):
        wid = lax.axis_index("s") * NC + lax.axis_index("c")
        base = wid * b_per_w
        pltpu.sync_copy(idx_hbm.at[wid], idx_v)

        def body(g, carry):
            pltpu.async_copy(table_hbm.at[idx_v.at[g]], rows_v, g_sem).wait()
            pltpu.sync_copy(rows_v, out_hbm.at[pl.ds(base + g * CHUNK, CHUNK)])
            return carry

        lax.fori_loop(0, n_chunks, body, 0)

    return lookup


def kernel(input_ids, emb_weight):
    b, l = input_ids.shape
    vocab, dim = emb_weight.shape
    n = b * l
    idx = input_ids.reshape(NW, n // (NW * CHUNK), CHUNK).astype(jnp.int32)
    table = jnp.pad(emb_weight, ((0, 0), (0, DPAD - dim)))
    out = _make_lookup(n)(idx, table)
    one = lax.optimization_barrier(jnp.float32(1.0))
    return (out[:, :dim] * one).reshape(b, l, dim)


# speedup vs baseline: 1.2143x; 1.2143x over previous
import functools

import jax
import jax.numpy as jnp
from jax import lax
from jax.experimental import pallas as pl
from jax.experimental.pallas import tpu as pltpu
from jax.experimental.pallas import tpu_sc as plsc

NC = 2
NS = 16
NW = NC * NS
CHUNK = 128
DPAD = 384


@functools.lru_cache(maxsize=None)
def _make_lookup(n_tokens: int):
    assert n_tokens % (NW * CHUNK) == 0
    b_per_w = n_tokens // NW
    n_chunks = b_per_w // CHUNK
    mesh = plsc.VectorSubcoreMesh(core_axis_name="c", subcore_axis_name="s")

    @functools.partial(
        pl.kernel,
        mesh=mesh,
        out_type=jax.ShapeDtypeStruct((n_tokens, DPAD), jnp.float32),
        scratch_types=[
            pltpu.VMEM((n_chunks, CHUNK), jnp.int32),
            pltpu.VMEM((CHUNK, DPAD), jnp.float32),
            pltpu.VMEM((CHUNK, DPAD), jnp.float32),
            pltpu.SemaphoreType.DMA,
            pltpu.SemaphoreType.DMA,
            pltpu.SemaphoreType.DMA,
            pltpu.SemaphoreType.DMA,
        ],
    )
    def lookup(idx_hbm, table_hbm, out_hbm, idx_v, rows0, rows1,
               g_sem0, g_sem1, s_sem0, s_sem1):
        wid = lax.axis_index("s") * NC + lax.axis_index("c")
        base = wid * b_per_w
        pltpu.sync_copy(idx_hbm.at[wid], idx_v)

        def out_at(g):
            return out_hbm.at[pl.ds(base + g * CHUNK, CHUNK)]

        def gather(g, buf, sem):
            pltpu.async_copy(table_hbm.at[idx_v.at[g]], buf, sem)

        gather(0, rows0, g_sem0)
        half = n_chunks // 2

        def body(gg, carry):
            e = 2 * gg
            gather(e + 1, rows1, g_sem1)
            pltpu.make_async_copy(table_hbm.at[idx_v.at[e]], rows0,
                                  g_sem0).wait()
            pltpu.async_copy(rows0, out_at(e), s_sem0)
            pltpu.make_async_copy(table_hbm.at[idx_v.at[e]], rows1,
                                  g_sem1).wait()
            pltpu.async_copy(rows1, out_at(e + 1), s_sem1)
            pltpu.make_async_copy(rows0, out_at(e), s_sem0).wait()

            @pl.when(gg != half - 1)
            def _():
                gather(e + 2, rows0, g_sem0)

            pltpu.make_async_copy(rows1, out_at(e + 1), s_sem1).wait()
            return carry

        lax.fori_loop(0, half, body, 0)

    return lookup


def kernel(input_ids, emb_weight):
    b, l = input_ids.shape
    vocab, dim = emb_weight.shape
    n = b * l
    idx = input_ids.reshape(NW, n // (NW * CHUNK), CHUNK).astype(jnp.int32)
    table = jnp.pad(emb_weight, ((0, 0), (0, DPAD - dim)))
    out = _make_lookup(n)(idx, table)
    return out[:, :dim].reshape(b, l, dim)
